# Initial kernel scaffold; baseline (speedup 1.0000x reference)
import jax
import jax.numpy as jnp
from jax.experimental import pallas as pl


def kernel(x, edge_index, W):
    raise NotImplementedError("write your pallas kernel here")


# speedup vs baseline: 16.0936x; 16.0936x over previous
import functools

import jax
import jax.numpy as jnp
from jax import lax
from jax.experimental import pallas as pl
from jax.experimental.pallas import tpu as pltpu
from jax.experimental.pallas import tpu_sc as plsc

N_NODES = 10000
D_FEAT = 128
UNITS = 128
N_EDGES = 320000

NC, NS, L = 2, 16, 16
NW = NC * NS
E_W = N_EDGES // NW
K = 80
NCHUNK = E_W // K
ROWS_W = 624
TAIL_ROWS = N_NODES - NS * ROWS_W

_MESH = plsc.VectorSubcoreMesh(core_axis_name="c", subcore_axis_name="s")
_SC_PARAMS = pltpu.CompilerParams(needs_layout_passes=False)


@functools.partial(
    pl.kernel,
    out_type=[
        jax.ShapeDtypeStruct((NW, N_NODES), jnp.float32),
        jax.ShapeDtypeStruct((NW, N_NODES), jnp.float32),
    ],
    mesh=_MESH,
    compiler_params=_SC_PARAMS,
    scratch_types=[
        pltpu.VMEM((E_W,), jnp.int32),
        pltpu.VMEM((E_W,), jnp.int32),
        pltpu.VMEM((N_NODES,), jnp.float32),
        pltpu.VMEM((N_NODES,), jnp.float32),
    ],
)
def _degrees(src_hbm, dst_hbm, odeg_hbm, ideg_hbm, src_v, dst_v, oh_v, ih_v):
    c = lax.axis_index("c")
    s = lax.axis_index("s")
    wid = c * NS + s
    pltpu.sync_copy(src_hbm.at[wid], src_v)
    pltpu.sync_copy(dst_hbm.at[wid], dst_v)

    zero = jnp.zeros((L,), jnp.float32)

    def zbody(i, carry):
        off = pl.multiple_of(i * L, L)
        oh_v[pl.ds(off, L)] = zero
        ih_v[pl.ds(off, L)] = zero
        return carry

    lax.fori_loop(0, N_NODES // L, zbody, 0)

    ones = jnp.ones((L,), jnp.float32)

    def body(i, carry):
        off = pl.multiple_of(i * L, L)
        si = src_v[pl.ds(off, L)]
        di = dst_v[pl.ds(off, L)]
        plsc.addupdate_scatter(oh_v, [si], ones)
        plsc.addupdate_scatter(ih_v, [di], ones)
        return carry

    lax.fori_loop(0, E_W // L, body, 0)

    pltpu.sync_copy(oh_v, odeg_hbm.at[wid])
    pltpu.sync_copy(ih_v, ideg_hbm.at[wid])


@functools.partial(
    pl.kernel,
    out_type=jax.ShapeDtypeStruct((NC, N_NODES, UNITS), jnp.float32),
    mesh=_MESH,
    compiler_params=_SC_PARAMS,
    scratch_types=[
        pltpu.VMEM((NCHUNK, K), jnp.int32),
        pltpu.VMEM((NCHUNK, K), jnp.int32),
        pltpu.VMEM((K, UNITS), jnp.float32),
        pltpu.VMEM_SHARED((N_NODES, UNITS), jnp.float32),
        pltpu.SemaphoreType.DMA,
    ],
)
def _segsum(y_hbm, src_hbm, dst_hbm, zeros_hbm, out_hbm, src_v, dst_v, rows_v, acc, sem):
    c = lax.axis_index("c")
    s = lax.axis_index("s")
    wid = c * NS + s
    pltpu.sync_copy(src_hbm.at[wid], src_v)
    pltpu.sync_copy(dst_hbm.at[wid], dst_v)

    r0 = s * ROWS_W
    pltpu.sync_copy(zeros_hbm, acc.at[pl.ds(r0, ROWS_W)])

    @pl.when(s == NS - 1)
    def _zero_tail():
        pltpu.sync_copy(
            zeros_hbm.at[pl.ds(0, TAIL_ROWS)],
            acc.at[pl.ds(NS * ROWS_W, TAIL_ROWS)],
        )

    plsc.subcore_barrier()

    def body(i, carry):
        pltpu.async_copy(y_hbm.at[src_v.at[i]], rows_v, sem).wait()
        pltpu.sync_copy(rows_v, acc.at[dst_v.at[i]], add=True)
        return carry

    lax.fori_loop(0, NCHUNK, body, 0)
    plsc.subcore_barrier()

    pltpu.sync_copy(acc.at[pl.ds(r0, ROWS_W)], out_hbm.at[c, pl.ds(r0, ROWS_W)])

    @pl.when(s == NS - 1)
    def _write_tail():
        pltpu.sync_copy(
            acc.at[pl.ds(NS * ROWS_W, TAIL_ROWS)],
            out_hbm.at[c, pl.ds(NS * ROWS_W, TAIL_ROWS)],
        )


_BLK = 1000
_GRID = N_NODES // _BLK


def _mm_body(od_ref, x_ref, w_ref, y_ref):
    deg = jnp.sum(od_ref[...], axis=1)
    sc = lax.rsqrt(jnp.maximum(deg, 1.0))
    y_ref[...] = jnp.dot(
        x_ref[...] * sc[:, None], w_ref[...], preferred_element_type=jnp.float32
    )


def _fin_body(p0_ref, p1_ref, id_ref, o_ref):
    deg = jnp.sum(id_ref[...], axis=1)
    sn = lax.rsqrt(jnp.maximum(deg, 1.0))
    o_ref[...] = jnp.maximum((p0_ref[...] + p1_ref[...]) * sn[:, None], 0.0)


def kernel(x, edge_index, W):
    src = edge_index[:, 0].astype(jnp.int32)
    dst = edge_index[:, 1].astype(jnp.int32)
    src_w = src.reshape(NW, E_W)
    dst_w = dst.reshape(NW, E_W)
    src3 = src.reshape(NW, NCHUNK, K)
    dst3 = dst.reshape(NW, NCHUNK, K)

    odeg_p, ideg_p = _degrees(src_w, dst_w)

    y = pl.pallas_call(
        _mm_body,
        grid=(_GRID,),
        in_specs=[
            pl.BlockSpec((_BLK, NW), lambda i: (i, 0)),
            pl.BlockSpec((_BLK, D_FEAT), lambda i: (i, 0)),
            pl.BlockSpec((D_FEAT, UNITS), lambda i: (0, 0)),
        ],
        out_specs=pl.BlockSpec((_BLK, UNITS), lambda i: (i, 0)),
        out_shape=jax.ShapeDtypeStruct((N_NODES, UNITS), jnp.float32),
    )(odeg_p.T, x, W)

    zeros = jnp.zeros((ROWS_W, UNITS), jnp.float32)
    partials = _segsum(y, src3, dst3, zeros)

    out = pl.pallas_call(
        _fin_body,
        grid=(_GRID,),
        in_specs=[
            pl.BlockSpec((_BLK, UNITS), lambda i: (i, 0)),
            pl.BlockSpec((_BLK, UNITS), lambda i: (i, 0)),
            pl.BlockSpec((_BLK, NW), lambda i: (i, 0)),
        ],
        out_specs=pl.BlockSpec((_BLK, UNITS), lambda i: (i, 0)),
        out_shape=jax.ShapeDtypeStruct((N_NODES, UNITS), jnp.float32),
    )(partials[0], partials[1], ideg_p.T)
    return out


# speedup vs baseline: 23.2125x; 1.4423x over previous
import functools

import jax
import jax.numpy as jnp
from jax import lax
from jax.experimental import pallas as pl
from jax.experimental.pallas import tpu as pltpu
from jax.experimental.pallas import tpu_sc as plsc

N_NODES = 10000
D_FEAT = 128
UNITS = 128
N_EDGES = 320000

NC, NS, L = 2, 16, 16
NW = NC * NS
E_W = N_EDGES // NW
K = 80
NCHUNK = E_W // K
ROWS_W = 624
TAIL_ROWS = N_NODES - NS * ROWS_W

_MESH = plsc.VectorSubcoreMesh(core_axis_name="c", subcore_axis_name="s")
_SC_PARAMS = pltpu.CompilerParams(needs_layout_passes=False)


@functools.partial(
    pl.kernel,
    out_type=[
        jax.ShapeDtypeStruct((NW, N_NODES), jnp.float32),
        jax.ShapeDtypeStruct((NW, N_NODES), jnp.float32),
    ],
    mesh=_MESH,
    compiler_params=_SC_PARAMS,
    scratch_types=[
        pltpu.VMEM((E_W,), jnp.int32),
        pltpu.VMEM((E_W,), jnp.int32),
        pltpu.VMEM((N_NODES,), jnp.float32),
        pltpu.VMEM((N_NODES,), jnp.float32),
    ],
)
def _degrees(src_hbm, dst_hbm, odeg_hbm, ideg_hbm, src_v, dst_v, oh_v, ih_v):
    c = lax.axis_index("c")
    s = lax.axis_index("s")
    wid = c * NS + s
    pltpu.sync_copy(src_hbm.at[wid], src_v)
    pltpu.sync_copy(dst_hbm.at[wid], dst_v)

    zero = jnp.zeros((L,), jnp.float32)

    def zbody(i, carry):
        off = pl.multiple_of(i * L, L)
        oh_v[pl.ds(off, L)] = zero
        ih_v[pl.ds(off, L)] = zero
        return carry

    lax.fori_loop(0, N_NODES // L, zbody, 0)

    ones = jnp.ones((L,), jnp.float32)

    def body(i, carry):
        off = pl.multiple_of(i * L, L)
        si = src_v[pl.ds(off, L)]
        di = dst_v[pl.ds(off, L)]
        plsc.addupdate_scatter(oh_v, [si], ones)
        plsc.addupdate_scatter(ih_v, [di], ones)
        return carry

    lax.fori_loop(0, E_W // L, body, 0)

    pltpu.sync_copy(oh_v, odeg_hbm.at[wid])
    pltpu.sync_copy(ih_v, ideg_hbm.at[wid])


@functools.partial(
    pl.kernel,
    out_type=jax.ShapeDtypeStruct((NC, N_NODES, UNITS), jnp.float32),
    mesh=_MESH,
    compiler_params=_SC_PARAMS,
    scratch_types=[
        pltpu.VMEM((NCHUNK, K), jnp.int32),
        pltpu.VMEM((K,), jnp.int32),
        pltpu.VMEM((K,), jnp.int32),
        pltpu.VMEM((K, UNITS), jnp.float32),
        pltpu.VMEM((K, UNITS), jnp.float32),
        pltpu.VMEM_SHARED((N_NODES, UNITS), jnp.float32),
        pltpu.SemaphoreType.DMA,
        pltpu.SemaphoreType.DMA,
        pltpu.SemaphoreType.DMA,
        pltpu.SemaphoreType.DMA,
    ],
)
def _segsum(y_hbm, src_hbm, dst_hbm, zeros_hbm, out_hbm,
            src_v, d0, d1, rows0, rows1, acc, sem0, sem1, dsem0, dsem1):
    c = lax.axis_index("c")
    s = lax.axis_index("s")
    wid = c * NS + s
    pltpu.sync_copy(src_hbm.at[wid], src_v)

    r0 = s * ROWS_W
    pltpu.sync_copy(zeros_hbm, acc.at[pl.ds(r0, ROWS_W)])

    @pl.when(s == NS - 1)
    def _zero_tail():
        pltpu.sync_copy(
            zeros_hbm.at[pl.ds(0, TAIL_ROWS)],
            acc.at[pl.ds(NS * ROWS_W, TAIL_ROWS)],
        )

    plsc.subcore_barrier()

    def start_gather(i, buf, sem):
        pltpu.async_copy(y_hbm.at[src_v.at[i]], buf, sem)

    def wait_gather(buf, sem):
        pltpu.make_async_copy(y_hbm.at[pl.ds(0, K)], buf, sem).wait()

    def start_didx(i, buf, sem):
        pltpu.async_copy(dst_hbm.at[wid, i], buf, sem)

    def wait_didx(buf, sem):
        pltpu.make_async_copy(dst_hbm.at[0, 0], buf, sem).wait()

    start_didx(0, d0, dsem0)
    start_gather(0, rows0, sem0)

    def body(j, carry):
        i = j * 2
        start_gather(i + 1, rows1, sem1)
        start_didx(i + 1, d1, dsem1)
        wait_didx(d0, dsem0)
        wait_gather(rows0, sem0)
        pltpu.sync_copy(rows0, acc.at[d0], add=True)
        start_gather(i + 2, rows0, sem0)
        start_didx(i + 2, d0, dsem0)
        wait_didx(d1, dsem1)
        wait_gather(rows1, sem1)
        pltpu.sync_copy(rows1, acc.at[d1], add=True)
        return carry

    lax.fori_loop(0, NCHUNK // 2, body, 0)
    wait_didx(d0, dsem0)
    wait_gather(rows0, sem0)
    pltpu.sync_copy(rows0, acc.at[d0], add=True)
    plsc.subcore_barrier()

    pltpu.sync_copy(acc.at[pl.ds(r0, ROWS_W)], out_hbm.at[c, pl.ds(r0, ROWS_W)])

    @pl.when(s == NS - 1)
    def _write_tail():
        pltpu.sync_copy(
            acc.at[pl.ds(NS * ROWS_W, TAIL_ROWS)],
            out_hbm.at[c, pl.ds(NS * ROWS_W, TAIL_ROWS)],
        )


_BLK = 1000
_GRID = N_NODES // _BLK


def _mm_body(od_ref, x_ref, w_ref, y_ref):
    deg = jnp.sum(od_ref[...], axis=1)
    sc = lax.rsqrt(jnp.maximum(deg, 1.0))
    y_ref[...] = jnp.dot(
        x_ref[...] * sc[:, None], w_ref[...], preferred_element_type=jnp.float32
    )


def _fin_body(p0_ref, p1_ref, id_ref, o_ref):
    deg = jnp.sum(id_ref[...], axis=1)
    sn = lax.rsqrt(jnp.maximum(deg, 1.0))
    o_ref[...] = jnp.maximum((p0_ref[...] + p1_ref[...]) * sn[:, None], 0.0)


def kernel(x, edge_index, W):
    src = edge_index[:, 0].astype(jnp.int32)
    dst = edge_index[:, 1].astype(jnp.int32)
    src_w = src.reshape(NW, E_W)
    dst_w = dst.reshape(NW, E_W)
    src3 = src.reshape(NW, NCHUNK, K)
    dst3 = dst.reshape(NW, NCHUNK, K)

    odeg_p, ideg_p = _degrees(src_w, dst_w)

    y = pl.pallas_call(
        _mm_body,
        grid=(_GRID,),
        in_specs=[
            pl.BlockSpec((_BLK, NW), lambda i: (i, 0)),
            pl.BlockSpec((_BLK, D_FEAT), lambda i: (i, 0)),
            pl.BlockSpec((D_FEAT, UNITS), lambda i: (0, 0)),
        ],
        out_specs=pl.BlockSpec((_BLK, UNITS), lambda i: (i, 0)),
        out_shape=jax.ShapeDtypeStruct((N_NODES, UNITS), jnp.float32),
    )(odeg_p.T, x, W)

    zeros = jnp.zeros((ROWS_W, UNITS), jnp.float32)
    partials = _segsum(y, src3, dst3, zeros)

    out = pl.pallas_call(
        _fin_body,
        grid=(_GRID,),
        in_specs=[
            pl.BlockSpec((_BLK, UNITS), lambda i: (i, 0)),
            pl.BlockSpec((_BLK, UNITS), lambda i: (i, 0)),
            pl.BlockSpec((_BLK, NW), lambda i: (i, 0)),
        ],
        out_specs=pl.BlockSpec((_BLK, UNITS), lambda i: (i, 0)),
        out_shape=jax.ShapeDtypeStruct((N_NODES, UNITS), jnp.float32),
    )(partials[0], partials[1], ideg_p.T)
    return out
